# baseline (speedup 1.0000x reference)
import functools

import jax
import jax.numpy as jnp
from jax import lax
from jax.experimental import pallas as pl
import jax.experimental.pallas.tpu as pltpu
from jax.experimental.pallas import tpu_sc as plsc

C = 384
QN, SN = 8, 16
HW = 64 * 64
Q_TOK = 8 * 16 * 16
S_TOK = 16 * 32 * 32 + 16 * 16 * 16
QB = 256
ST = 2048
CHUNK = 128
N_CHUNKS = S_TOK // CHUNK
KEEP = 16
ROWS_PER_W = Q_TOK // 32


def _pool_maps():
    hw = jnp.arange(HW)
    h, w = hw // 64, hw % 64
    tok_q = (h // 4) * 16 + (w // 4)
    tok_1 = (h // 2) * 32 + (w // 2)
    pq = (tok_q[:, None] == jnp.arange(256)[None, :]).astype(jnp.bfloat16) * jnp.bfloat16(1 / 16)
    p1 = (tok_1[:, None] == jnp.arange(1024)[None, :]).astype(jnp.bfloat16) * jnp.bfloat16(1 / 4)
    hw2 = jnp.arange(1024)
    h2, w2 = hw2 // 32, hw2 % 32
    tok_2 = (h2 // 2) * 16 + (w2 // 2)
    p2 = (tok_2[:, None] == jnp.arange(256)[None, :]).astype(jnp.bfloat16) * jnp.bfloat16(1 / 4)
    return pq, p1, p2


def _pool_q_body(x_ref, pq_ref, out_ref):
    x = x_ref[0].astype(jnp.bfloat16)
    t = jnp.dot(x, pq_ref[...], preferred_element_type=jnp.float32)
    n2 = jnp.sum(t * t, axis=0, keepdims=True)
    out_ref[...] = (t * jax.lax.rsqrt(n2)).astype(jnp.bfloat16)


def _pool_s_body(x_ref, p1_ref, p2_ref, o1_ref, o2_ref):
    x = x_ref[0].astype(jnp.bfloat16)
    t1 = jnp.dot(x, p1_ref[...], preferred_element_type=jnp.float32)
    t2 = jnp.dot(t1.astype(jnp.bfloat16), p2_ref[...],
                 preferred_element_type=jnp.float32)
    n1 = jnp.sum(t1 * t1, axis=0, keepdims=True)
    n2 = jnp.sum(t2 * t2, axis=0, keepdims=True)
    o1_ref[...] = (t1 * jax.lax.rsqrt(n1)).astype(jnp.bfloat16)
    o2_ref[...] = (t2 * jax.lax.rsqrt(n2)).astype(jnp.bfloat16)


def _main_body(qn_ref, sn_ref, sc_ref, g_ref):
    sc = jax.lax.dot_general(
        qn_ref[...], sn_ref[...],
        dimension_numbers=(((0,), (0,)), ((), ())),
        preferred_element_type=jnp.float32)
    sc_ref[...] = sc
    cms = [jnp.max(sc[:, k * CHUNK:(k + 1) * CHUNK], axis=1, keepdims=True)
           for k in range(ST // CHUNK)]
    g_ref[...] = jnp.concatenate(cms, axis=1).reshape(1, QB, ST // CHUNK)


def _sc_gather_body(g_hbm, sc_hbm, out_hbm, g_v, idx_v, rows_v, sem):
    wid = lax.axis_index("s") * 2 + lax.axis_index("c")
    base = wid * ROWS_PER_W
    pltpu.sync_copy(g_hbm.at[pl.ds(base, ROWS_PER_W)], g_v)
    iota16 = lax.iota(jnp.int32, 16)

    def body(i, carry):
        tk, ti = plsc.sort_key_val(g_v[i, pl.ds(0, 16)], iota16)
        for k in range(1, N_CHUNKS // 16):
            vk, vi = plsc.sort_key_val(g_v[i, pl.ds(k * 16, 16)],
                                       iota16 + k * 16, descending=True)
            b = vk > tk
            nk = jnp.where(b, vk, tk)
            ni = jnp.where(b, vi, ti)
            tk, ti = plsc.sort_key_val(nk, ni)
        row = base + i
        idx_v[...] = ti + row * N_CHUNKS
        pltpu.async_copy(sc_hbm.at[idx_v], rows_v, sem).wait()
        pltpu.sync_copy(rows_v, out_hbm.at[row])
        return carry

    lax.fori_loop(0, ROWS_PER_W, body, 0)


def _extract_body(x_ref, out_ref):
    qb = pl.program_id(0)
    neg_inf = jnp.float32(-jnp.inf)
    v0 = jnp.max(x_ref[...], axis=1, keepdims=True)

    def sweep(r, carry):
        v, s15, s4 = carry
        below = jnp.where(x_ref[...] < v, x_ref[...], neg_inf)
        nxt = jnp.max(below, axis=1, keepdims=True)
        vs = jnp.maximum(v, -4.0)
        s15 = s15 + jnp.exp(vs - v0)
        s4 = s4 + jnp.where(r < 4, vs, 0.0)
        return (nxt, s15, s4)

    zero = jnp.zeros((QB, 1), jnp.float32)
    v14, s15, s4 = jax.lax.fori_loop(0, 14, sweep, (v0, zero, zero))
    s15 = s15 + jnp.exp(jnp.maximum(v14, -4.0) - v0)
    loss_rows = jnp.log(s15) - 0.25 * (s4 - 4.0 * v0)
    partial = (jnp.sum(loss_rows) / jnp.float32(Q_TOK)).reshape(1, 1)

    @pl.when(qb == 0)
    def _():
        out_ref[...] = partial

    @pl.when(qb != 0)
    def _():
        out_ref[...] = out_ref[...] + partial


@jax.jit
def kernel(q, S):
    pq, p1, p2 = _pool_maps()
    q3 = q.reshape(QN, C, HW)
    s3 = S.reshape(SN, C, HW)

    qn = pl.pallas_call(
        _pool_q_body,
        grid=(QN,),
        in_specs=[
            pl.BlockSpec((1, C, HW), lambda n: (n, 0, 0)),
            pl.BlockSpec((HW, 256), lambda n: (0, 0)),
        ],
        out_specs=pl.BlockSpec((C, 256), lambda n: (0, n)),
        out_shape=jax.ShapeDtypeStruct((C, Q_TOK), jnp.bfloat16),
    )(q3, pq)

    s1n, s2n = pl.pallas_call(
        _pool_s_body,
        grid=(SN,),
        in_specs=[
            pl.BlockSpec((1, C, HW), lambda n: (n, 0, 0)),
            pl.BlockSpec((HW, 1024), lambda n: (0, 0)),
            pl.BlockSpec((1024, 256), lambda n: (0, 0)),
        ],
        out_specs=[
            pl.BlockSpec((C, 1024), lambda n: (0, n)),
            pl.BlockSpec((C, 256), lambda n: (0, n)),
        ],
        out_shape=[
            jax.ShapeDtypeStruct((C, SN * 1024), jnp.bfloat16),
            jax.ShapeDtypeStruct((C, SN * 256), jnp.bfloat16),
        ],
    )(s3, p1, p2)

    sn = jnp.concatenate([s1n, s2n], axis=1)

    scores, g3 = pl.pallas_call(
        _main_body,
        grid=(Q_TOK // QB, S_TOK // ST),
        in_specs=[
            pl.BlockSpec((C, QB), lambda qb, st: (0, qb)),
            pl.BlockSpec((C, ST), lambda qb, st: (0, st)),
        ],
        out_specs=[
            pl.BlockSpec((QB, ST), lambda qb, st: (qb, st)),
            pl.BlockSpec((1, QB, ST // CHUNK), lambda qb, st: (st, qb, 0)),
        ],
        out_shape=[
            jax.ShapeDtypeStruct((Q_TOK, S_TOK), jnp.float32),
            jax.ShapeDtypeStruct((S_TOK // ST, Q_TOK, ST // CHUNK), jnp.float32),
        ],
    )(qn, sn)

    g = jnp.transpose(g3, (1, 0, 2)).reshape(Q_TOK, N_CHUNKS)
    sc_table = scores.reshape(Q_TOK * N_CHUNKS, CHUNK)

    gather = functools.partial(
        pl.kernel,
        mesh=plsc.VectorSubcoreMesh(core_axis_name="c", subcore_axis_name="s"),
        out_type=jax.ShapeDtypeStruct((Q_TOK, KEEP, CHUNK), jnp.float32),
        scratch_types=[
            pltpu.VMEM((ROWS_PER_W, N_CHUNKS), jnp.float32),
            pltpu.VMEM((KEEP,), jnp.int32),
            pltpu.VMEM((KEEP, CHUNK), jnp.float32),
            pltpu.SemaphoreType.DMA,
        ],
        compiler_params=pltpu.CompilerParams(needs_layout_passes=False),
    )(_sc_gather_body)
    compact = gather(g, sc_table).reshape(Q_TOK, KEEP * CHUNK)

    out = pl.pallas_call(
        _extract_body,
        grid=(Q_TOK // QB,),
        in_specs=[pl.BlockSpec((QB, KEEP * CHUNK), lambda qb: (qb, 0))],
        out_specs=pl.BlockSpec((1, 1), lambda qb: (0, 0)),
        out_shape=jax.ShapeDtypeStruct((1, 1), jnp.float32),
    )(compact)
    return out[0, 0]


# speedup vs baseline: 1.0234x; 1.0234x over previous
import functools

import jax
import jax.numpy as jnp
from jax import lax
from jax.experimental import pallas as pl
import jax.experimental.pallas.tpu as pltpu
from jax.experimental.pallas import tpu_sc as plsc

C = 384
QN, SN = 8, 16
HW = 64 * 64
Q_TOK = 8 * 16 * 16
S_TOK = 16 * 32 * 32 + 16 * 16 * 16
QB = 256
ST = 2048
CHUNK = 128
N_CHUNKS = S_TOK // CHUNK
KEEP = 16
ROWS_PER_W = Q_TOK // 32


def _pool_maps():
    hw = jnp.arange(HW)
    h, w = hw // 64, hw % 64
    tok_q = (h // 4) * 16 + (w // 4)
    tok_1 = (h // 2) * 32 + (w // 2)
    pq = (tok_q[:, None] == jnp.arange(256)[None, :]).astype(jnp.bfloat16) * jnp.bfloat16(1 / 16)
    p1 = (tok_1[:, None] == jnp.arange(1024)[None, :]).astype(jnp.bfloat16) * jnp.bfloat16(1 / 4)
    hw2 = jnp.arange(1024)
    h2, w2 = hw2 // 32, hw2 % 32
    tok_2 = (h2 // 2) * 16 + (w2 // 2)
    p2 = (tok_2[:, None] == jnp.arange(256)[None, :]).astype(jnp.bfloat16) * jnp.bfloat16(1 / 4)
    return pq, p1, p2


def _pool_body(xq_ref, xs_ref, pq_ref, p1_ref, p2_ref, qn_ref, o1_ref, o2_ref):
    n = pl.program_id(0)
    xs = xs_ref[0].astype(jnp.bfloat16)
    t1 = jnp.dot(xs, p1_ref[...], preferred_element_type=jnp.float32)
    t2 = jnp.dot(t1.astype(jnp.bfloat16), p2_ref[...],
                 preferred_element_type=jnp.float32)
    n1 = jnp.sum(t1 * t1, axis=0, keepdims=True)
    n2 = jnp.sum(t2 * t2, axis=0, keepdims=True)
    o1_ref[...] = (t1 * jax.lax.rsqrt(n1)).astype(jnp.bfloat16)
    o2_ref[...] = (t2 * jax.lax.rsqrt(n2)).astype(jnp.bfloat16)

    @pl.when(n < QN)
    def _():
        xq = xq_ref[0].astype(jnp.bfloat16)
        t = jnp.dot(xq, pq_ref[...], preferred_element_type=jnp.float32)
        nq = jnp.sum(t * t, axis=0, keepdims=True)
        qn_ref[...] = (t * jax.lax.rsqrt(nq)).astype(jnp.bfloat16)


def _main_body(qn_ref, sn_ref, sc_ref, g_ref, g_scr):
    st = pl.program_id(1)
    sc = jax.lax.dot_general(
        qn_ref[...], sn_ref[...],
        dimension_numbers=(((0,), (0,)), ((), ())),
        preferred_element_type=jnp.float32)
    sc_ref[...] = sc
    cms = [jnp.max(sc[:, k * CHUNK:(k + 1) * CHUNK], axis=1, keepdims=True)
           for k in range(ST // CHUNK)]
    g_scr[st] = jnp.concatenate(cms, axis=1)

    @pl.when(st == (S_TOK // ST) - 1)
    def _():
        g_ref[...] = jnp.concatenate(
            [g_scr[i] for i in range(S_TOK // ST)], axis=1)


def _sc_gather_body(g_hbm, sc_hbm, out_hbm, g_v, idx_v, rows_v, sem):
    wid = lax.axis_index("s") * 2 + lax.axis_index("c")
    base = wid * ROWS_PER_W
    pltpu.sync_copy(g_hbm.at[pl.ds(base, ROWS_PER_W)], g_v)
    iota16 = lax.iota(jnp.int32, 16)

    def body(i, carry):
        tk, ti = plsc.sort_key_val(g_v[i, pl.ds(0, 16)], iota16)
        for k in range(1, N_CHUNKS // 16):
            vk, vi = plsc.sort_key_val(g_v[i, pl.ds(k * 16, 16)],
                                       iota16 + k * 16, descending=True)
            b = vk > tk
            nk = jnp.where(b, vk, tk)
            ni = jnp.where(b, vi, ti)
            tk, ti = plsc.sort_key_val(nk, ni)
        row = base + i
        idx_v[...] = ti + row * N_CHUNKS
        pltpu.async_copy(sc_hbm.at[idx_v], rows_v, sem).wait()
        pltpu.sync_copy(rows_v, out_hbm.at[row])
        return carry

    lax.fori_loop(0, ROWS_PER_W, body, 0)


def _extract_body(x_ref, out_ref):
    qb = pl.program_id(0)
    neg_inf = jnp.float32(-jnp.inf)
    v0 = jnp.max(x_ref[...], axis=1, keepdims=True)

    def sweep(r, carry):
        v, s15, s4 = carry
        below = jnp.where(x_ref[...] < v, x_ref[...], neg_inf)
        nxt = jnp.max(below, axis=1, keepdims=True)
        vs = jnp.maximum(v, -4.0)
        s15 = s15 + jnp.exp(vs - v0)
        s4 = s4 + jnp.where(r < 4, vs, 0.0)
        return (nxt, s15, s4)

    zero = jnp.zeros((QB, 1), jnp.float32)
    v14, s15, s4 = jax.lax.fori_loop(0, 14, sweep, (v0, zero, zero))
    s15 = s15 + jnp.exp(jnp.maximum(v14, -4.0) - v0)
    loss_rows = jnp.log(s15) - 0.25 * (s4 - 4.0 * v0)
    partial = (jnp.sum(loss_rows) / jnp.float32(Q_TOK)).reshape(1, 1)

    @pl.when(qb == 0)
    def _():
        out_ref[...] = partial

    @pl.when(qb != 0)
    def _():
        out_ref[...] = out_ref[...] + partial


@jax.jit
def kernel(q, S):
    pq, p1, p2 = _pool_maps()
    q3 = q.reshape(QN, C, HW)
    s3 = S.reshape(SN, C, HW)

    qn, s1n, s2n = pl.pallas_call(
        _pool_body,
        grid=(SN,),
        in_specs=[
            pl.BlockSpec((1, C, HW), lambda n: (jnp.minimum(n, QN - 1), 0, 0)),
            pl.BlockSpec((1, C, HW), lambda n: (n, 0, 0)),
            pl.BlockSpec((HW, 256), lambda n: (0, 0)),
            pl.BlockSpec((HW, 1024), lambda n: (0, 0)),
            pl.BlockSpec((1024, 256), lambda n: (0, 0)),
        ],
        out_specs=[
            pl.BlockSpec((C, 256), lambda n: (0, jnp.minimum(n, QN - 1))),
            pl.BlockSpec((C, 1024), lambda n: (0, n)),
            pl.BlockSpec((C, 256), lambda n: (0, n)),
        ],
        out_shape=[
            jax.ShapeDtypeStruct((C, Q_TOK), jnp.bfloat16),
            jax.ShapeDtypeStruct((C, SN * 1024), jnp.bfloat16),
            jax.ShapeDtypeStruct((C, SN * 256), jnp.bfloat16),
        ],
    )(q3, s3, pq, p1, p2)

    sn = jnp.concatenate([s1n, s2n], axis=1)

    scores, g = pl.pallas_call(
        _main_body,
        grid=(Q_TOK // QB, S_TOK // ST),
        in_specs=[
            pl.BlockSpec((C, QB), lambda qb, st: (0, qb)),
            pl.BlockSpec((C, ST), lambda qb, st: (0, st)),
        ],
        out_specs=[
            pl.BlockSpec((QB, ST), lambda qb, st: (qb, st)),
            pl.BlockSpec((QB, N_CHUNKS), lambda qb, st: (qb, 0)),
        ],
        out_shape=[
            jax.ShapeDtypeStruct((Q_TOK, S_TOK), jnp.float32),
            jax.ShapeDtypeStruct((Q_TOK, N_CHUNKS), jnp.float32),
        ],
        scratch_shapes=[pltpu.VMEM((S_TOK // ST, QB, ST // CHUNK), jnp.float32)],
    )(qn, sn)

    sc_table = scores.reshape(Q_TOK * N_CHUNKS, CHUNK)

    gather = functools.partial(
        pl.kernel,
        mesh=plsc.VectorSubcoreMesh(core_axis_name="c", subcore_axis_name="s"),
        out_type=jax.ShapeDtypeStruct((Q_TOK, KEEP, CHUNK), jnp.float32),
        scratch_types=[
            pltpu.VMEM((ROWS_PER_W, N_CHUNKS), jnp.float32),
            pltpu.VMEM((KEEP,), jnp.int32),
            pltpu.VMEM((KEEP, CHUNK), jnp.float32),
            pltpu.SemaphoreType.DMA,
        ],
        compiler_params=pltpu.CompilerParams(needs_layout_passes=False),
    )(_sc_gather_body)
    compact = gather(g, sc_table).reshape(Q_TOK, KEEP * CHUNK)

    out = pl.pallas_call(
        _extract_body,
        grid=(Q_TOK // QB,),
        in_specs=[pl.BlockSpec((QB, KEEP * CHUNK), lambda qb: (qb, 0))],
        out_specs=pl.BlockSpec((1, 1), lambda qb: (0, 0)),
        out_shape=jax.ShapeDtypeStruct((1, 1), jnp.float32),
    )(compact)
    return out[0, 0]


# speedup vs baseline: 1.2748x; 1.2456x over previous
import functools

import jax
import jax.numpy as jnp
from jax import lax
from jax.experimental import pallas as pl
import jax.experimental.pallas.tpu as pltpu
from jax.experimental.pallas import tpu_sc as plsc

C = 384
QN, SN = 8, 16
HW = 64 * 64
Q_TOK = 8 * 16 * 16
S_TOK = 16 * 32 * 32 + 16 * 16 * 16
QB = 256
ST = 2048
CHUNK = 128
N_CHUNKS = S_TOK // CHUNK
KEEP = 16
ROWS_PER_W = Q_TOK // 32


def _pool_maps():
    hw = jnp.arange(HW)
    h, w = hw // 64, hw % 64
    tok_q = (h // 4) * 16 + (w // 4)
    tok_1 = (h // 2) * 32 + (w // 2)
    pq = (tok_q[:, None] == jnp.arange(256)[None, :]).astype(jnp.bfloat16) * jnp.bfloat16(1 / 16)
    p1 = (tok_1[:, None] == jnp.arange(1024)[None, :]).astype(jnp.bfloat16) * jnp.bfloat16(1 / 4)
    hw2 = jnp.arange(1024)
    h2, w2 = hw2 // 32, hw2 % 32
    tok_2 = (h2 // 2) * 16 + (w2 // 2)
    p2 = (tok_2[:, None] == jnp.arange(256)[None, :]).astype(jnp.bfloat16) * jnp.bfloat16(1 / 4)
    return pq, p1, p2


def _pool_body(xq_ref, xs_ref, pq_ref, p1_ref, p2_ref, qn_ref, o1_ref, o2_ref):
    n = pl.program_id(0)
    xs = xs_ref[0].astype(jnp.bfloat16)
    t1 = jnp.dot(xs, p1_ref[...], preferred_element_type=jnp.float32)
    t2 = jnp.dot(t1.astype(jnp.bfloat16), p2_ref[...],
                 preferred_element_type=jnp.float32)
    n1 = jnp.sum(t1 * t1, axis=0, keepdims=True)
    n2 = jnp.sum(t2 * t2, axis=0, keepdims=True)
    o1_ref[...] = (t1 * jax.lax.rsqrt(n1)).astype(jnp.bfloat16)
    o2_ref[...] = (t2 * jax.lax.rsqrt(n2)).astype(jnp.bfloat16)

    @pl.when(n < QN)
    def _():
        xq = xq_ref[0].astype(jnp.bfloat16)
        t = jnp.dot(xq, pq_ref[...], preferred_element_type=jnp.float32)
        nq = jnp.sum(t * t, axis=0, keepdims=True)
        qn_ref[...] = (t * jax.lax.rsqrt(nq)).astype(jnp.bfloat16)


def _main_body(qn_ref, sn_ref, sc_ref, g_ref, g_scr):
    st = pl.program_id(1)
    sc = jax.lax.dot_general(
        qn_ref[...], sn_ref[...],
        dimension_numbers=(((0,), (0,)), ((), ())),
        preferred_element_type=jnp.float32)
    sc_ref[...] = sc.reshape(QB, ST // CHUNK, CHUNK)
    cms = [jnp.max(sc[:, k * CHUNK:(k + 1) * CHUNK], axis=1, keepdims=True)
           for k in range(ST // CHUNK)]
    g_scr[st] = jnp.concatenate(cms, axis=1)

    @pl.when(st == (S_TOK // ST) - 1)
    def _():
        g_ref[...] = jnp.concatenate(
            [g_scr[i] for i in range(S_TOK // ST)], axis=1)


def _sc_gather_body(g_hbm, sc_hbm, out_hbm, g_v, idx_v, rows_v, sem):
    wid = lax.axis_index("s") * 2 + lax.axis_index("c")
    base = wid * ROWS_PER_W
    pltpu.sync_copy(g_hbm.at[pl.ds(base, ROWS_PER_W)], g_v)
    iota16 = lax.iota(jnp.int32, 16)

    def body(i, carry):
        tk, ti = plsc.sort_key_val(g_v[i, pl.ds(0, 16)], iota16)
        for k in range(1, N_CHUNKS // 16):
            vk, vi = plsc.sort_key_val(g_v[i, pl.ds(k * 16, 16)],
                                       iota16 + k * 16, descending=True)
            b = vk > tk
            nk = jnp.where(b, vk, tk)
            ni = jnp.where(b, vi, ti)
            tk, ti = plsc.sort_key_val(nk, ni)
        row = base + i
        idx_v[...] = ti + row * N_CHUNKS
        pltpu.async_copy(sc_hbm.at[idx_v], rows_v, sem).wait()
        pltpu.sync_copy(rows_v, out_hbm.at[row])
        return carry

    lax.fori_loop(0, ROWS_PER_W, body, 0)


def _extract_body(x_ref, out_ref):
    qb = pl.program_id(0)
    neg_inf = jnp.float32(-jnp.inf)
    v0 = jnp.max(jnp.max(x_ref[...], axis=2), axis=1, keepdims=True)

    def sweep(r, carry):
        v, s15, s4 = carry
        below = jnp.where(x_ref[...] < v[:, :, None], x_ref[...], neg_inf)
        nxt = jnp.max(jnp.max(below, axis=2), axis=1, keepdims=True)
        vs = jnp.maximum(v, -4.0)
        s15 = s15 + jnp.exp(vs - v0)
        s4 = s4 + jnp.where(r < 4, vs, 0.0)
        return (nxt, s15, s4)

    zero = jnp.zeros((QB, 1), jnp.float32)
    v14, s15, s4 = jax.lax.fori_loop(0, 14, sweep, (v0, zero, zero))
    s15 = s15 + jnp.exp(jnp.maximum(v14, -4.0) - v0)
    loss_rows = jnp.log(s15) - 0.25 * (s4 - 4.0 * v0)
    partial = (jnp.sum(loss_rows) / jnp.float32(Q_TOK)).reshape(1, 1)

    @pl.when(qb == 0)
    def _():
        out_ref[...] = partial

    @pl.when(qb != 0)
    def _():
        out_ref[...] = out_ref[...] + partial


@jax.jit
def kernel(q, S):
    pq, p1, p2 = _pool_maps()
    q3 = q.reshape(QN, C, HW)
    s3 = S.reshape(SN, C, HW)

    qn, s1n, s2n = pl.pallas_call(
        _pool_body,
        grid=(SN,),
        in_specs=[
            pl.BlockSpec((1, C, HW), lambda n: (jnp.minimum(n, QN - 1), 0, 0)),
            pl.BlockSpec((1, C, HW), lambda n: (n, 0, 0)),
            pl.BlockSpec((HW, 256), lambda n: (0, 0)),
            pl.BlockSpec((HW, 1024), lambda n: (0, 0)),
            pl.BlockSpec((1024, 256), lambda n: (0, 0)),
        ],
        out_specs=[
            pl.BlockSpec((C, 256), lambda n: (0, jnp.minimum(n, QN - 1))),
            pl.BlockSpec((C, 1024), lambda n: (0, n)),
            pl.BlockSpec((C, 256), lambda n: (0, n)),
        ],
        out_shape=[
            jax.ShapeDtypeStruct((C, Q_TOK), jnp.bfloat16),
            jax.ShapeDtypeStruct((C, SN * 1024), jnp.bfloat16),
            jax.ShapeDtypeStruct((C, SN * 256), jnp.bfloat16),
        ],
    )(q3, s3, pq, p1, p2)

    sn = jnp.concatenate([s1n, s2n], axis=1)

    scores, g = pl.pallas_call(
        _main_body,
        grid=(Q_TOK // QB, S_TOK // ST),
        in_specs=[
            pl.BlockSpec((C, QB), lambda qb, st: (0, qb)),
            pl.BlockSpec((C, ST), lambda qb, st: (0, st)),
        ],
        out_specs=[
            pl.BlockSpec((QB, ST // CHUNK, CHUNK), lambda qb, st: (qb, st, 0)),
            pl.BlockSpec((QB, N_CHUNKS), lambda qb, st: (qb, 0)),
        ],
        out_shape=[
            jax.ShapeDtypeStruct((Q_TOK, N_CHUNKS, CHUNK), jnp.float32),
            jax.ShapeDtypeStruct((Q_TOK, N_CHUNKS), jnp.float32),
        ],
        scratch_shapes=[pltpu.VMEM((S_TOK // ST, QB, ST // CHUNK), jnp.float32)],
    )(qn, sn)

    sc_table = scores.reshape(Q_TOK * N_CHUNKS, CHUNK)

    gather = functools.partial(
        pl.kernel,
        mesh=plsc.VectorSubcoreMesh(core_axis_name="c", subcore_axis_name="s"),
        out_type=jax.ShapeDtypeStruct((Q_TOK, KEEP, CHUNK), jnp.float32),
        scratch_types=[
            pltpu.VMEM((ROWS_PER_W, N_CHUNKS), jnp.float32),
            pltpu.VMEM((KEEP,), jnp.int32),
            pltpu.VMEM((KEEP, CHUNK), jnp.float32),
            pltpu.SemaphoreType.DMA,
        ],
        compiler_params=pltpu.CompilerParams(needs_layout_passes=False),
    )(_sc_gather_body)
    compact = gather(g, sc_table)

    out = pl.pallas_call(
        _extract_body,
        grid=(Q_TOK // QB,),
        in_specs=[pl.BlockSpec((QB, KEEP, CHUNK), lambda qb: (qb, 0, 0))],
        out_specs=pl.BlockSpec((1, 1), lambda qb: (0, 0)),
        out_shape=jax.ShapeDtypeStruct((1, 1), jnp.float32),
    )(compact)
    return out[0, 0]


# speedup vs baseline: 1.3339x; 1.0464x over previous
import functools

import jax
import jax.numpy as jnp
from jax import lax
from jax.experimental import pallas as pl
import jax.experimental.pallas.tpu as pltpu
from jax.experimental.pallas import tpu_sc as plsc

C = 384
QN, SN = 8, 16
HW = 64 * 64
Q_TOK = 8 * 16 * 16
S_TOK = 16 * 32 * 32 + 16 * 16 * 16
QB = 256
ST = 4096
CHUNK = 128
N_CHUNKS = S_TOK // CHUNK
KEEP = 16
ROWS_PER_W = Q_TOK // 32


def _pool_maps():
    hw = jnp.arange(HW)
    h, w = hw // 64, hw % 64
    tok_q = (h // 4) * 16 + (w // 4)
    tok_1 = (h // 2) * 32 + (w // 2)
    pq = (tok_q[:, None] == jnp.arange(256)[None, :]).astype(jnp.bfloat16) * jnp.bfloat16(1 / 16)
    p1 = (tok_1[:, None] == jnp.arange(1024)[None, :]).astype(jnp.bfloat16) * jnp.bfloat16(1 / 4)
    hw2 = jnp.arange(1024)
    h2, w2 = hw2 // 32, hw2 % 32
    tok_2 = (h2 // 2) * 16 + (w2 // 2)
    p2 = (tok_2[:, None] == jnp.arange(256)[None, :]).astype(jnp.bfloat16) * jnp.bfloat16(1 / 4)
    return pq, p1, p2


def _pool_body(xq_ref, xs_ref, pq_ref, p1_ref, p2_ref, qn_ref, o1_ref, o2_ref):
    n = pl.program_id(0)
    xs = xs_ref[0].astype(jnp.bfloat16)
    t1 = jnp.dot(xs, p1_ref[...], preferred_element_type=jnp.float32)
    t2 = jnp.dot(t1.astype(jnp.bfloat16), p2_ref[...],
                 preferred_element_type=jnp.float32)
    n1 = jnp.sum(t1 * t1, axis=0, keepdims=True)
    n2 = jnp.sum(t2 * t2, axis=0, keepdims=True)
    o1_ref[...] = (t1 * jax.lax.rsqrt(n1)).astype(jnp.bfloat16)
    o2_ref[...] = (t2 * jax.lax.rsqrt(n2)).astype(jnp.bfloat16)

    @pl.when(n < QN)
    def _():
        xq = xq_ref[0].astype(jnp.bfloat16)
        t = jnp.dot(xq, pq_ref[...], preferred_element_type=jnp.float32)
        nq = jnp.sum(t * t, axis=0, keepdims=True)
        qn_ref[...] = (t * jax.lax.rsqrt(nq)).astype(jnp.bfloat16)


def _main_body(qn_ref, sn_ref, sc_ref, g_ref, g_scr):
    st = pl.program_id(1)
    sc = jax.lax.dot_general(
        qn_ref[...], sn_ref[...],
        dimension_numbers=(((0,), (0,)), ((), ())),
        preferred_element_type=jnp.float32)
    sc_ref[...] = sc.reshape(QB, ST // CHUNK, CHUNK)
    cms = [jnp.max(sc[:, k * CHUNK:(k + 1) * CHUNK], axis=1, keepdims=True)
           for k in range(ST // CHUNK)]
    g_scr[st] = jnp.concatenate(cms, axis=1)

    @pl.when(st == (S_TOK // ST) - 1)
    def _():
        g_ref[...] = jnp.concatenate(
            [g_scr[i] for i in range(S_TOK // ST)], axis=1)


def _sc_gather_body(g_hbm, sc_hbm, out_hbm, g_v, idx_v, rows_v, sem):
    wid = lax.axis_index("s") * 2 + lax.axis_index("c")
    base = wid * ROWS_PER_W
    pltpu.sync_copy(g_hbm.at[pl.ds(base, ROWS_PER_W)], g_v)
    iota16 = lax.iota(jnp.int32, 16)

    def body(i, carry):
        tk, ti = plsc.sort_key_val(g_v[i, pl.ds(0, 16)], iota16)
        for k in range(1, N_CHUNKS // 16):
            vk, vi = plsc.sort_key_val(g_v[i, pl.ds(k * 16, 16)],
                                       iota16 + k * 16, descending=True)
            b = vk > tk
            nk = jnp.where(b, vk, tk)
            ni = jnp.where(b, vi, ti)
            tk, ti = plsc.sort_key_val(nk, ni)
        row = base + i
        idx_v[...] = ti + row * N_CHUNKS
        pltpu.async_copy(sc_hbm.at[idx_v], rows_v, sem).wait()
        pltpu.sync_copy(rows_v, out_hbm.at[row])
        return carry

    lax.fori_loop(0, ROWS_PER_W, body, 0)


def _extract_body(x_ref, out_ref):
    qb = pl.program_id(0)
    neg_inf = jnp.float32(-jnp.inf)
    v0 = jnp.max(jnp.max(x_ref[...], axis=2), axis=1, keepdims=True)

    def sweep(r, carry):
        v, s15, s4 = carry
        below = jnp.where(x_ref[...] < v[:, :, None], x_ref[...], neg_inf)
        nxt = jnp.max(jnp.max(below, axis=2), axis=1, keepdims=True)
        vs = jnp.maximum(v, -4.0)
        s15 = s15 + jnp.exp(vs - v0)
        s4 = s4 + jnp.where(r < 4, vs, 0.0)
        return (nxt, s15, s4)

    zero = jnp.zeros((QB, 1), jnp.float32)
    v14, s15, s4 = jax.lax.fori_loop(0, 14, sweep, (v0, zero, zero))
    s15 = s15 + jnp.exp(jnp.maximum(v14, -4.0) - v0)
    loss_rows = jnp.log(s15) - 0.25 * (s4 - 4.0 * v0)
    partial = (jnp.sum(loss_rows) / jnp.float32(Q_TOK)).reshape(1, 1)

    @pl.when(qb == 0)
    def _():
        out_ref[...] = partial

    @pl.when(qb != 0)
    def _():
        out_ref[...] = out_ref[...] + partial


@jax.jit
def kernel(q, S):
    pq, p1, p2 = _pool_maps()
    q3 = q.reshape(QN, C, HW)
    s3 = S.reshape(SN, C, HW)

    qn, s1n, s2n = pl.pallas_call(
        _pool_body,
        grid=(SN,),
        in_specs=[
            pl.BlockSpec((1, C, HW), lambda n: (jnp.minimum(n, QN - 1), 0, 0)),
            pl.BlockSpec((1, C, HW), lambda n: (n, 0, 0)),
            pl.BlockSpec((HW, 256), lambda n: (0, 0)),
            pl.BlockSpec((HW, 1024), lambda n: (0, 0)),
            pl.BlockSpec((1024, 256), lambda n: (0, 0)),
        ],
        out_specs=[
            pl.BlockSpec((C, 256), lambda n: (0, jnp.minimum(n, QN - 1))),
            pl.BlockSpec((C, 1024), lambda n: (0, n)),
            pl.BlockSpec((C, 256), lambda n: (0, n)),
        ],
        out_shape=[
            jax.ShapeDtypeStruct((C, Q_TOK), jnp.bfloat16),
            jax.ShapeDtypeStruct((C, SN * 1024), jnp.bfloat16),
            jax.ShapeDtypeStruct((C, SN * 256), jnp.bfloat16),
        ],
    )(q3, s3, pq, p1, p2)

    sn = jnp.concatenate([s1n, s2n], axis=1)

    scores, g = pl.pallas_call(
        _main_body,
        grid=(Q_TOK // QB, S_TOK // ST),
        in_specs=[
            pl.BlockSpec((C, QB), lambda qb, st: (0, qb)),
            pl.BlockSpec((C, ST), lambda qb, st: (0, st)),
        ],
        out_specs=[
            pl.BlockSpec((QB, ST // CHUNK, CHUNK), lambda qb, st: (qb, st, 0)),
            pl.BlockSpec((QB, N_CHUNKS), lambda qb, st: (qb, 0)),
        ],
        out_shape=[
            jax.ShapeDtypeStruct((Q_TOK, N_CHUNKS, CHUNK), jnp.float32),
            jax.ShapeDtypeStruct((Q_TOK, N_CHUNKS), jnp.float32),
        ],
        scratch_shapes=[pltpu.VMEM((S_TOK // ST, QB, ST // CHUNK), jnp.float32)],
    )(qn, sn)

    sc_table = scores.reshape(Q_TOK * N_CHUNKS, CHUNK)

    gather = functools.partial(
        pl.kernel,
        mesh=plsc.VectorSubcoreMesh(core_axis_name="c", subcore_axis_name="s"),
        out_type=jax.ShapeDtypeStruct((Q_TOK, KEEP, CHUNK), jnp.float32),
        scratch_types=[
            pltpu.VMEM((ROWS_PER_W, N_CHUNKS), jnp.float32),
            pltpu.VMEM((KEEP,), jnp.int32),
            pltpu.VMEM((KEEP, CHUNK), jnp.float32),
            pltpu.SemaphoreType.DMA,
        ],
        compiler_params=pltpu.CompilerParams(needs_layout_passes=False),
    )(_sc_gather_body)
    compact = gather(g, sc_table)

    out = pl.pallas_call(
        _extract_body,
        grid=(Q_TOK // QB,),
        in_specs=[pl.BlockSpec((QB, KEEP, CHUNK), lambda qb: (qb, 0, 0))],
        out_specs=pl.BlockSpec((1, 1), lambda qb: (0, 0)),
        out_shape=jax.ShapeDtypeStruct((1, 1), jnp.float32),
    )(compact)
    return out[0, 0]
